# baseline (speedup 1.0000x reference)
import jax
import jax.numpy as jnp
from jax import lax
from jax.experimental import pallas as pl
from jax.experimental.pallas import tpu as pltpu

_SB = 4096


def _tc_body(half_ref, w_ref, x_hbm, pos_hbm, sel_hbm, tgt_hbm,
             xv, posv, selv, tgtv, sem_x, sem_p, sem_o):
    i = pl.program_id(0)
    j = pl.program_id(1)
    nb = pl.num_programs(1)
    k = i * nb + j
    total = pl.num_programs(0) * nb
    slot = jax.lax.rem(k, 2)
    nxt = jax.lax.rem(k + 1, 2)

    def start_in(i2, j2, buf):
        pltpu.make_async_copy(
            x_hbm.at[j2, :, pl.ds(i2 * _SB, _SB)], xv.at[buf], sem_x.at[buf]
        ).start()
        pltpu.make_async_copy(
            pos_hbm.at[j2, pl.ds(i2 * _SB, _SB)], posv.at[buf], sem_p.at[buf]
        ).start()

    @pl.when(k == 0)
    def _():
        start_in(0, 0, 0)

    @pl.when(k + 1 < total)
    def _():
        jn = j + 1
        start_in(i + jn // nb, jax.lax.rem(jn, nb), nxt)

    pltpu.make_async_copy(
        x_hbm.at[j, :, pl.ds(i * _SB, _SB)], xv.at[slot], sem_x.at[slot]
    ).wait()
    pltpu.make_async_copy(
        pos_hbm.at[j, pl.ds(i * _SB, _SB)], posv.at[slot], sem_p.at[slot]
    ).wait()

    @pl.when(k >= 2)
    def _():
        kp = k - 2
        ip = kp // nb
        jp = jax.lax.rem(kp, nb)
        pltpu.make_async_copy(
            selv.at[slot], sel_hbm.at[jp, pl.ds(ip * _SB, _SB)],
            sem_o.at[slot, 0]).wait()
        pltpu.make_async_copy(
            tgtv.at[slot], tgt_hbm.at[jp, pl.ds(ip * _SB, _SB)],
            sem_o.at[slot, 1]).wait()

    w = w_ref[...]
    half = half_ref[0]
    xb = xv[slot]
    st = lax.dot_general(w, xb, (((1,), (0,)), ((), ())),
                         preferred_element_type=jnp.float32)
    best = st[0:1, :]
    arg = jnp.zeros_like(best, dtype=jnp.int32)
    for t in range(1, 8):
        row = st[t:t + 1, :]
        m = row > best
        best = jnp.where(m, row, best)
        arg = jnp.where(m, t, arg)
    selv[slot] = arg.reshape(-1)
    pos = posv[slot]
    x0 = xv[slot, 0]
    x1 = xv[slot, 1]
    tgtv[slot] = (jnp.where(pos >= half, 4, 0) + jnp.where(x0 > 0, 2, 0)
                  + jnp.where(x1 > 0, 1, 0)).astype(jnp.int32)

    pltpu.make_async_copy(
        selv.at[slot], sel_hbm.at[j, pl.ds(i * _SB, _SB)],
        sem_o.at[slot, 0]).start()
    pltpu.make_async_copy(
        tgtv.at[slot], tgt_hbm.at[j, pl.ds(i * _SB, _SB)],
        sem_o.at[slot, 1]).start()

    @pl.when(k == total - 1)
    def _():
        other = jax.lax.rem(k + 1, 2)

        @pl.when(total >= 2)
        def _():
            kp = k - 1
            ip = kp // nb
            jp = jax.lax.rem(kp, nb)
            pltpu.make_async_copy(
                selv.at[other], sel_hbm.at[jp, pl.ds(ip * _SB, _SB)],
                sem_o.at[other, 0]).wait()
            pltpu.make_async_copy(
                tgtv.at[other], tgt_hbm.at[jp, pl.ds(ip * _SB, _SB)],
                sem_o.at[other, 1]).wait()

        pltpu.make_async_copy(
            selv.at[slot], sel_hbm.at[j, pl.ds(i * _SB, _SB)],
            sem_o.at[slot, 0]).wait()
        pltpu.make_async_copy(
            tgtv.at[slot], tgt_hbm.at[j, pl.ds(i * _SB, _SB)],
            sem_o.at[slot, 1]).wait()


def kernel(x, positions, seq_len, content_sigs):
    b, s, d = x.shape
    w = jnp.tanh(content_sigs)
    half = ((jnp.asarray(seq_len) + 1) // 2).astype(jnp.int32).reshape(1)
    pos = positions.astype(jnp.int32)
    xt = jnp.transpose(x, (0, 2, 1))
    xt = pltpu.with_memory_space_constraint(xt, pltpu.MemorySpace.HBM)
    pos = pltpu.with_memory_space_constraint(pos, pltpu.MemorySpace.HBM)

    grid = (s // _SB, b)
    sel, tgt = pl.pallas_call(
        _tc_body,
        grid=grid,
        in_specs=[
            pl.BlockSpec(memory_space=pltpu.SMEM),
            pl.BlockSpec((8, d), lambda i, j: (0, 0)),
            pl.BlockSpec(memory_space=pltpu.MemorySpace.HBM),
            pl.BlockSpec(memory_space=pltpu.MemorySpace.HBM),
        ],
        out_specs=[
            pl.BlockSpec(memory_space=pltpu.MemorySpace.HBM),
            pl.BlockSpec(memory_space=pltpu.MemorySpace.HBM),
        ],
        out_shape=[
            jax.ShapeDtypeStruct((b, s), jnp.int32),
            jax.ShapeDtypeStruct((b, s), jnp.int32),
        ],
        scratch_shapes=[
            pltpu.VMEM((2, d, _SB), jnp.float32),
            pltpu.VMEM((2, _SB), jnp.int32),
            pltpu.VMEM((2, _SB), jnp.int32),
            pltpu.VMEM((2, _SB), jnp.int32),
            pltpu.SemaphoreType.DMA((2,)),
            pltpu.SemaphoreType.DMA((2,)),
            pltpu.SemaphoreType.DMA((2, 2)),
        ],
        compiler_params=pltpu.CompilerParams(
            dimension_semantics=("arbitrary", "arbitrary")),
    )(half, w, xt, pos)
    return sel, tgt


# speedup vs baseline: 1.6312x; 1.6312x over previous
import jax
import jax.numpy as jnp
from jax import lax
from jax.experimental import pallas as pl
from jax.experimental.pallas import tpu as pltpu

_SB = 4096


def _tc_body(half_ref, w_ref, x_hbm, pos_hbm, sel_hbm, tgt_hbm,
             xv, posv, selv, tgtv, sem_x, sem_p, sem_o):
    i = pl.program_id(0)
    nsteps = pl.num_programs(0)
    slot = jax.lax.rem(i, 2)
    nxt = jax.lax.rem(i + 1, 2)

    def start_in(j, buf):
        pltpu.make_async_copy(
            x_hbm.at[:, :, pl.ds(j * _SB, _SB)], xv.at[buf], sem_x.at[buf]
        ).start()
        pltpu.make_async_copy(
            pos_hbm.at[:, pl.ds(j * _SB, _SB)], posv.at[buf], sem_p.at[buf]
        ).start()

    @pl.when(i == 0)
    def _():
        start_in(0, 0)

    @pl.when(i + 1 < nsteps)
    def _():
        start_in(i + 1, nxt)

    pltpu.make_async_copy(
        x_hbm.at[:, :, pl.ds(i * _SB, _SB)], xv.at[slot], sem_x.at[slot]
    ).wait()
    pltpu.make_async_copy(
        pos_hbm.at[:, pl.ds(i * _SB, _SB)], posv.at[slot], sem_p.at[slot]
    ).wait()

    @pl.when(i >= 2)
    def _():
        pltpu.make_async_copy(
            selv.at[slot], sel_hbm.at[:, pl.ds((i - 2) * _SB, _SB)],
            sem_o.at[slot, 0]).wait()
        pltpu.make_async_copy(
            tgtv.at[slot], tgt_hbm.at[:, pl.ds((i - 2) * _SB, _SB)],
            sem_o.at[slot, 1]).wait()

    w = jnp.tanh(w_ref[...])
    half = half_ref[0]
    nb = xv.shape[1]
    for b in range(nb):
        xb = xv[slot, b]
        st = lax.dot_general(w, xb, (((1,), (0,)), ((), ())),
                             preferred_element_type=jnp.float32)
        best = st[0:1, :]
        arg = jnp.zeros_like(best, dtype=jnp.int32)
        for t in range(1, 8):
            row = st[t:t + 1, :]
            m = row > best
            best = jnp.where(m, row, best)
            arg = jnp.where(m, t, arg)
        selv[slot, b] = arg.reshape(-1)
        pos = posv[slot, b]
        x0 = xv[slot, b, 0]
        x1 = xv[slot, b, 1]
        tgtv[slot, b] = (jnp.where(pos >= half, 4, 0) + jnp.where(x0 > 0, 2, 0)
                         + jnp.where(x1 > 0, 1, 0)).astype(jnp.int32)

    pltpu.make_async_copy(
        selv.at[slot], sel_hbm.at[:, pl.ds(i * _SB, _SB)],
        sem_o.at[slot, 0]).start()
    pltpu.make_async_copy(
        tgtv.at[slot], tgt_hbm.at[:, pl.ds(i * _SB, _SB)],
        sem_o.at[slot, 1]).start()

    @pl.when(i == nsteps - 1)
    def _():
        other = jax.lax.rem(i + 1, 2)

        @pl.when(nsteps >= 2)
        def _():
            pltpu.make_async_copy(
                selv.at[other], sel_hbm.at[:, pl.ds((i - 1) * _SB, _SB)],
                sem_o.at[other, 0]).wait()
            pltpu.make_async_copy(
                tgtv.at[other], tgt_hbm.at[:, pl.ds((i - 1) * _SB, _SB)],
                sem_o.at[other, 1]).wait()

        pltpu.make_async_copy(
            selv.at[slot], sel_hbm.at[:, pl.ds(i * _SB, _SB)],
            sem_o.at[slot, 0]).wait()
        pltpu.make_async_copy(
            tgtv.at[slot], tgt_hbm.at[:, pl.ds(i * _SB, _SB)],
            sem_o.at[slot, 1]).wait()


def kernel(x, positions, seq_len, content_sigs):
    b, s, d = x.shape
    half = ((jnp.asarray(seq_len) + 1) // 2).astype(jnp.int32).reshape(1)
    pos = positions.astype(jnp.int32)
    xt = jnp.transpose(x, (0, 2, 1))

    grid = (s // _SB,)
    sel, tgt = pl.pallas_call(
        _tc_body,
        grid=grid,
        in_specs=[
            pl.BlockSpec(memory_space=pltpu.SMEM),
            pl.BlockSpec((8, d), lambda i: (0, 0)),
            pl.BlockSpec(memory_space=pltpu.MemorySpace.HBM),
            pl.BlockSpec(memory_space=pltpu.MemorySpace.HBM),
        ],
        out_specs=[
            pl.BlockSpec(memory_space=pltpu.MemorySpace.HBM),
            pl.BlockSpec(memory_space=pltpu.MemorySpace.HBM),
        ],
        out_shape=[
            jax.ShapeDtypeStruct((b, s), jnp.int32),
            jax.ShapeDtypeStruct((b, s), jnp.int32),
        ],
        scratch_shapes=[
            pltpu.VMEM((2, b, d, _SB), jnp.float32),
            pltpu.VMEM((2, b, _SB), jnp.int32),
            pltpu.VMEM((2, b, _SB), jnp.int32),
            pltpu.VMEM((2, b, _SB), jnp.int32),
            pltpu.SemaphoreType.DMA((2,)),
            pltpu.SemaphoreType.DMA((2,)),
            pltpu.SemaphoreType.DMA((2, 2)),
        ],
        compiler_params=pltpu.CompilerParams(
            dimension_semantics=("arbitrary",)),
    )(half, content_sigs,
      pltpu.with_memory_space_constraint(xt, pltpu.MemorySpace.HBM),
      pltpu.with_memory_space_constraint(pos, pltpu.MemorySpace.HBM))
    return sel, tgt
